# baseline (speedup 1.0000x reference)
import functools

import jax
import jax.numpy as jnp
from jax import lax
from jax.experimental import pallas as pl
from jax.experimental.pallas import tpu as pltpu
from jax.experimental.pallas import tpu_sc as plsc

N = 10000
NP = 10240
E = 320000
D_IN, D_HID, D_OUT = 128, 256, 128
HQ = 128

NC, NS = 2, 16
K = 128
SUP = 8
IRS = 2
NRING = 2
RPT = NP // NS

BN = 1000
NBLK = N // BN


def _sc_mesh():
    return plsc.VectorSubcoreMesh(
        core_axis_name="c", subcore_axis_name="s", num_cores=NC, num_subcores=NS
    )


def _pad_tile_rows(a, fill, ept_p):
    t, ept = a.shape
    pad = ept_p - ept
    a = jnp.concatenate(
        [a, jnp.full((t, pad), fill, jnp.int32)], axis=1)
    return a.reshape(t, ept_p // (SUP * K), SUP, K)


def _edge_body(y_hbm, srcT, dstT, zrows_hbm, out_hbm, sring, dring, buf, acc,
               issem, idsem, gsem, ssem, *, nsup):
    c = lax.axis_index("c")
    s = lax.axis_index("s")
    w = c * NS + s
    nchp = nsup * SUP

    pltpu.sync_copy(zrows_hbm, acc.at[pl.ds(s * RPT, RPT)])

    for r in range(IRS):
        pltpu.async_copy(srcT.at[w].at[r], sring.at[r], issem.at[r])
        pltpu.async_copy(dstT.at[w].at[r], dring.at[r], idsem.at[r])
    plsc.subcore_barrier()

    pltpu.make_async_copy(srcT.at[w].at[0], sring.at[0], issem.at[0]).wait()
    pltpu.async_copy(y_hbm.at[sring.at[0].at[0]], buf.at[0], gsem.at[0])

    def super_body(u, _):
        su = lax.rem(u, IRS)
        pltpu.make_async_copy(dstT.at[w].at[u], dring.at[su], idsem.at[su]).wait()
        for j in range(SUP):
            i = u * SUP + j
            m = j % NRING
            m1 = (j + 1) % NRING
            pltpu.make_async_copy(y_hbm.at[sring.at[su].at[j]], buf.at[m],
                                  gsem.at[m]).wait()
            pltpu.async_copy(buf.at[m], acc.at[dring.at[su].at[j]],
                             ssem.at[m], add=True)

            def _drain_prev():
                pltpu.make_async_copy(buf.at[m1], acc.at[dring.at[su].at[j]],
                                      ssem.at[m1]).wait()
            if j > 0:
                _drain_prev()
            else:
                pl.when(u > 0)(_drain_prev)

                @pl.when(jnp.logical_and(u > 0, u + 1 < nsup))
                def _():
                    sn = lax.rem(u + 1, IRS)
                    pltpu.async_copy(srcT.at[w].at[u + 1], sring.at[sn],
                                     issem.at[sn])
                    pltpu.async_copy(dstT.at[w].at[u + 1], dring.at[sn],
                                     idsem.at[sn])

            if j == SUP - 1:
                @pl.when(u + 1 < nsup)
                def _():
                    pltpu.make_async_copy(srcT.at[w].at[u + 1],
                                          sring.at[lax.rem(u + 1, IRS)],
                                          issem.at[lax.rem(u + 1, IRS)]).wait()

            nxt = i + 1
            jn = (j + 1) % SUP

            @pl.when(nxt < nchp)
            def _():
                un = lax.rem((u + 1) if j == SUP - 1 else u, IRS)
                pltpu.async_copy(y_hbm.at[sring.at[un].at[jn]],
                                 buf.at[m1], gsem.at[m1])

        return 0

    lax.fori_loop(0, nsup, super_body, 0)
    mlast = (nchp - 1) % NRING
    pltpu.make_async_copy(buf.at[mlast], acc.at[dring.at[0].at[0]],
                          ssem.at[mlast]).wait()
    plsc.subcore_barrier()

    pltpu.sync_copy(acc.at[pl.ds(s * RPT, RPT)],
                    out_hbm.at[pl.ds(c * NP + s * RPT, RPT)])


def _edge_aggregate(y_table, srcT, dstT):
    nsup = srcT.shape[1]
    kern = pl.kernel(
        functools.partial(_edge_body, nsup=nsup),
        out_type=jax.ShapeDtypeStruct((2 * NP, HQ), jnp.float32),
        mesh=_sc_mesh(),
        scratch_types=[
            pltpu.VMEM((IRS, SUP, K), jnp.int32),
            pltpu.VMEM((IRS, SUP, K), jnp.int32),
            pltpu.VMEM((NRING, K, HQ), jnp.float32),
            pltpu.VMEM_SHARED((NP, HQ), jnp.float32),
            pltpu.SemaphoreType.DMA((IRS,)),
            pltpu.SemaphoreType.DMA((IRS,)),
            pltpu.SemaphoreType.DMA((NRING,)),
            pltpu.SemaphoreType.DMA((NRING,)),
        ],
    )
    zrows = jnp.zeros((RPT, HQ), jnp.float32)
    return kern(y_table, srcT, dstT, zrows)


def _deg_body(dstT, ones_hbm, zrows_hbm, deg_hbm, dring, ones_v, acc, idsem,
              *, nsup):
    c = lax.axis_index("c")
    s = lax.axis_index("s")
    w = c * NS + s

    pltpu.sync_copy(zrows_hbm, acc.at[pl.ds(s * RPT, RPT)])
    pltpu.sync_copy(ones_hbm, ones_v)
    for r in range(IRS):
        pltpu.async_copy(dstT.at[w].at[r], dring.at[r], idsem.at[r])
    plsc.subcore_barrier()

    def super_body(u, _):
        su = lax.rem(u, IRS)
        pltpu.make_async_copy(dstT.at[w].at[u], dring.at[su], idsem.at[su]).wait()
        for j in range(SUP):
            pltpu.sync_copy(ones_v, acc.at[dring.at[su].at[j]], add=True)

        @pl.when(u + IRS < nsup)
        def _():
            pltpu.async_copy(dstT.at[w].at[u + IRS], dring.at[su], idsem.at[su])
        return 0

    lax.fori_loop(0, nsup, super_body, 0)
    plsc.subcore_barrier()

    pltpu.sync_copy(acc.at[pl.ds(s * RPT, RPT)],
                    deg_hbm.at[pl.ds(c * NP + s * RPT, RPT)])


def _deg_counts(dstT):
    nsup = dstT.shape[1]
    kern = pl.kernel(
        functools.partial(_deg_body, nsup=nsup),
        out_type=jax.ShapeDtypeStruct((2 * NP, HQ), jnp.float32),
        mesh=_sc_mesh(),
        scratch_types=[
            pltpu.VMEM((IRS, SUP, K), jnp.int32),
            pltpu.VMEM((K, HQ), jnp.float32),
            pltpu.VMEM_SHARED((NP, HQ), jnp.float32),
            pltpu.SemaphoreType.DMA((IRS,)),
        ],
    )
    ones = jnp.ones((K, HQ), jnp.float32)
    zrows = jnp.zeros((RPT, HQ), jnp.float32)
    return kern(dstT, ones, zrows)


def _dinv_block(dg_ref):
    deg = dg_ref[0, :, 0:1] + dg_ref[1, :, 0:1] + 1.0
    return lax.rsqrt(deg)


def _mm_first_body(x_ref, w_ref, dg_ref, y_ref):
    dinv = _dinv_block(dg_ref)
    full = dinv * jnp.dot(x_ref[:, :], w_ref[:, :],
                          preferred_element_type=jnp.float32)
    y_ref[0, :, :] = full[:, :HQ]
    y_ref[1, :, :] = full[:, HQ:]


def _mm_first(x, W, deg):
    return pl.pallas_call(
        _mm_first_body,
        grid=(NBLK,),
        in_specs=[
            pl.BlockSpec((BN, D_IN), lambda n: (n, 0)),
            pl.BlockSpec((D_IN, D_HID), lambda n: (0, 0)),
            pl.BlockSpec((2, BN, HQ), lambda n: (0, n, 0)),
        ],
        out_specs=pl.BlockSpec((2, BN, HQ), lambda n: (0, n, 0)),
        out_shape=jax.ShapeDtypeStruct((2, NP, HQ), jnp.float32),
    )(x, W, deg)


def _ep_mm_body(y_ref, s_ref, dg_ref, b_ref, w_ref, out_ref):
    dinv = _dinv_block(dg_ref)
    z = jnp.concatenate([y_ref[0] + s_ref[0], y_ref[1] + s_ref[1]], axis=1)
    h = jnp.maximum(z * dinv + b_ref[:, :], 0.0)
    full = dinv * jnp.dot(h, w_ref[:, :], preferred_element_type=jnp.float32)
    if out_ref.shape[0] == 2:
        out_ref[0, :, :] = full[:, :HQ]
        out_ref[1, :, :] = full[:, HQ:]
    else:
        out_ref[:, :] = full


def _ep_mm(y, sagg, deg, b, W):
    d_out = W.shape[1]
    if d_out == 2 * HQ:
        out_spec = pl.BlockSpec((2, BN, HQ), lambda n: (0, n, 0))
        out_shape = jax.ShapeDtypeStruct((2, NP, HQ), jnp.float32)
    else:
        out_spec = pl.BlockSpec((BN, d_out), lambda n: (n, 0))
        out_shape = jax.ShapeDtypeStruct((NP, d_out), jnp.float32)
    return pl.pallas_call(
        _ep_mm_body,
        grid=(NBLK,),
        in_specs=[
            pl.BlockSpec((2, BN, HQ), lambda n: (0, n, 0)),
            pl.BlockSpec((2, BN, HQ), lambda n: (0, n, 0)),
            pl.BlockSpec((2, BN, HQ), lambda n: (0, n, 0)),
            pl.BlockSpec((1, 2 * HQ), lambda n: (0, 0)),
            pl.BlockSpec((2 * HQ, d_out), lambda n: (0, 0)),
        ],
        out_specs=out_spec,
        out_shape=out_shape,
    )(y, sagg, deg, b.reshape(1, -1), W)


def _final_body(y_ref, s_ref, dg_ref, b_ref, out_ref):
    dinv = _dinv_block(dg_ref)
    z = y_ref[:, :] + s_ref[0] + s_ref[1]
    out_ref[:, :] = z * dinv + b_ref[:, :]


def _final(y, sagg, deg, b):
    d_out = y.shape[1]
    return pl.pallas_call(
        _final_body,
        grid=(NBLK,),
        in_specs=[
            pl.BlockSpec((BN, d_out), lambda n: (n, 0)),
            pl.BlockSpec((2, BN, d_out), lambda n: (0, n, 0)),
            pl.BlockSpec((2, BN, HQ), lambda n: (0, n, 0)),
            pl.BlockSpec((1, d_out), lambda n: (0, 0)),
        ],
        out_specs=pl.BlockSpec((BN, d_out), lambda n: (n, 0)),
        out_shape=jax.ShapeDtypeStruct((N, d_out), jnp.float32),
    )(y, sagg, deg, b.reshape(1, -1))


def kernel(x, edge_index, W1, b1, W2, b2, W3, b3):
    ei = edge_index.astype(jnp.int32)
    src, dst = ei[0], ei[1]

    ept12 = E // NS
    ept12_p = 20480
    src_t = src.reshape(NS, ept12)
    dst_t = dst.reshape(NS, ept12)
    srcT12 = _pad_tile_rows(
        jnp.concatenate([src_t, src_t + NP], axis=0), 0, ept12_p)
    dstT12 = _pad_tile_rows(
        jnp.concatenate([dst_t, dst_t], axis=0), N, ept12_p)

    ept3 = E // (NC * NS)
    ept3_p = 10240
    srcT3 = _pad_tile_rows(src.reshape(NC * NS, ept3), 0, ept3_p)
    dstT3 = _pad_tile_rows(dst.reshape(NC * NS, ept3), N, ept3_p)

    deg = _deg_counts(dstT3).reshape(2, NP, HQ)

    y1 = _mm_first(x, W1, deg)
    s1 = _edge_aggregate(y1.reshape(2 * NP, HQ), srcT12, dstT12)
    y2 = _ep_mm(y1, s1.reshape(2, NP, HQ), deg, b1, W2)
    s2 = _edge_aggregate(y2.reshape(2 * NP, HQ), srcT12, dstT12)
    y3 = _ep_mm(y2, s2.reshape(2, NP, HQ), deg, b2, W3)
    s3 = _edge_aggregate(y3, srcT3, dstT3)
    return _final(y3, s3.reshape(2, NP, HQ), deg, b3)


# speedup vs baseline: 1.2849x; 1.2849x over previous
import functools

import jax
import jax.numpy as jnp
from jax import lax
from jax.experimental import pallas as pl
from jax.experimental.pallas import tpu as pltpu
from jax.experimental.pallas import tpu_sc as plsc

N = 10000
NP = 10240
E = 320000
D_IN, D_HID, D_OUT = 128, 256, 128
HQ = 128

NC, NS = 2, 16
K = 128
SUP = 8
IRS = 2
NRING = 2
RPT = NP // NS

BN = 1000
NBLK = N // BN


def _sc_mesh():
    return plsc.VectorSubcoreMesh(
        core_axis_name="c", subcore_axis_name="s", num_cores=NC, num_subcores=NS
    )


def _pad_tile_rows(a, fill, ept_p):
    t, ept = a.shape
    pad = ept_p - ept
    a = jnp.concatenate(
        [a, jnp.full((t, pad), fill, jnp.int32)], axis=1)
    return a.reshape(t, ept_p // (SUP * K), SUP, K)


def _edge_body(y_hbm, srcT, dstT, zrows_hbm, out_hbm, sring, dring, buf, acc,
               issem, idsem, gsem, *, nsup):
    c = lax.axis_index("c")
    s = lax.axis_index("s")
    w = c * NS + s
    nchp = nsup * SUP

    pltpu.sync_copy(zrows_hbm, acc.at[pl.ds(s * RPT, RPT)])

    for r in range(IRS):
        pltpu.async_copy(srcT.at[w].at[r], sring.at[r], issem.at[r])
        pltpu.async_copy(dstT.at[w].at[r], dring.at[r], idsem.at[r])
    plsc.subcore_barrier()

    pltpu.make_async_copy(srcT.at[w].at[0], sring.at[0], issem.at[0]).wait()
    for b in range(NRING):
        pltpu.async_copy(y_hbm.at[sring.at[0].at[b]], buf.at[b], gsem.at[b])

    def super_body(u, _):
        su = lax.rem(u, IRS)
        pltpu.make_async_copy(dstT.at[w].at[u], dring.at[su], idsem.at[su]).wait()
        for j in range(SUP):
            i = u * SUP + j
            m = j % NRING
            pltpu.make_async_copy(y_hbm.at[sring.at[su].at[j]], buf.at[m],
                                  gsem.at[m]).wait()
            pltpu.sync_copy(buf.at[m], acc.at[dring.at[su].at[j]], add=True)

            if j == SUP - NRING:
                @pl.when(u + 1 < nsup)
                def _():
                    pltpu.make_async_copy(srcT.at[w].at[u + 1],
                                          sring.at[lax.rem(u + 1, IRS)],
                                          issem.at[lax.rem(u + 1, IRS)]).wait()

            nxt = i + NRING
            un, jn = nxt // SUP, nxt % SUP

            @pl.when(nxt < nchp)
            def _():
                pltpu.async_copy(y_hbm.at[sring.at[lax.rem(un, IRS)].at[jn]],
                                 buf.at[m], gsem.at[m])

        @pl.when(u + IRS < nsup)
        def _():
            pltpu.async_copy(srcT.at[w].at[u + IRS], sring.at[su],
                             issem.at[su])
            pltpu.async_copy(dstT.at[w].at[u + IRS], dring.at[su],
                             idsem.at[su])
        return 0

    lax.fori_loop(0, nsup, super_body, 0)
    plsc.subcore_barrier()

    pltpu.sync_copy(acc.at[pl.ds(s * RPT, RPT)],
                    out_hbm.at[pl.ds(c * NP + s * RPT, RPT)])


def _edge_aggregate(y_table, srcT, dstT):
    nsup = srcT.shape[1]
    kern = pl.kernel(
        functools.partial(_edge_body, nsup=nsup),
        out_type=jax.ShapeDtypeStruct((2 * NP, HQ), jnp.float32),
        mesh=_sc_mesh(),
        scratch_types=[
            pltpu.VMEM((IRS, SUP, K), jnp.int32),
            pltpu.VMEM((IRS, SUP, K), jnp.int32),
            pltpu.VMEM((NRING, K, HQ), jnp.float32),
            pltpu.VMEM_SHARED((NP, HQ), jnp.float32),
            pltpu.SemaphoreType.DMA((IRS,)),
            pltpu.SemaphoreType.DMA((IRS,)),
            pltpu.SemaphoreType.DMA((NRING,)),
        ],
    )
    zrows = jnp.zeros((RPT, HQ), jnp.float32)
    return kern(y_table, srcT, dstT, zrows)


def _deg_body(dstT, ones_hbm, zrows_hbm, deg_hbm, dring, ones_v, acc, idsem,
              *, nsup):
    c = lax.axis_index("c")
    s = lax.axis_index("s")
    w = c * NS + s

    pltpu.sync_copy(zrows_hbm, acc.at[pl.ds(s * RPT, RPT)])
    pltpu.sync_copy(ones_hbm, ones_v)
    for r in range(IRS):
        pltpu.async_copy(dstT.at[w].at[r], dring.at[r], idsem.at[r])
    plsc.subcore_barrier()

    def super_body(u, _):
        su = lax.rem(u, IRS)
        pltpu.make_async_copy(dstT.at[w].at[u], dring.at[su], idsem.at[su]).wait()
        for j in range(SUP):
            pltpu.sync_copy(ones_v, acc.at[dring.at[su].at[j]], add=True)

        @pl.when(u + IRS < nsup)
        def _():
            pltpu.async_copy(dstT.at[w].at[u + IRS], dring.at[su], idsem.at[su])
        return 0

    lax.fori_loop(0, nsup, super_body, 0)
    plsc.subcore_barrier()

    pltpu.sync_copy(acc.at[pl.ds(s * RPT, RPT)],
                    deg_hbm.at[pl.ds(c * NP + s * RPT, RPT)])


def _deg_counts(dstT):
    nsup = dstT.shape[1]
    kern = pl.kernel(
        functools.partial(_deg_body, nsup=nsup),
        out_type=jax.ShapeDtypeStruct((2 * NP, HQ), jnp.float32),
        mesh=_sc_mesh(),
        scratch_types=[
            pltpu.VMEM((IRS, SUP, K), jnp.int32),
            pltpu.VMEM((K, HQ), jnp.float32),
            pltpu.VMEM_SHARED((NP, HQ), jnp.float32),
            pltpu.SemaphoreType.DMA((IRS,)),
        ],
    )
    ones = jnp.ones((K, HQ), jnp.float32)
    zrows = jnp.zeros((RPT, HQ), jnp.float32)
    return kern(dstT, ones, zrows)


def _dinv_block(dg_ref):
    deg = dg_ref[0, :, 0:1] + dg_ref[1, :, 0:1] + 1.0
    return lax.rsqrt(deg)


def _gx_body(x_ref, dg_ref, g_ref):
    g_ref[:, :] = _dinv_block(dg_ref) * x_ref[:, :]


def _gx(x, deg):
    return pl.pallas_call(
        _gx_body,
        grid=(NBLK,),
        in_specs=[
            pl.BlockSpec((BN, D_IN), lambda n: (n, 0)),
            pl.BlockSpec((2, BN, HQ), lambda n: (0, n, 0)),
        ],
        out_specs=pl.BlockSpec((BN, D_IN), lambda n: (n, 0)),
        out_shape=jax.ShapeDtypeStruct((NP, D_IN), jnp.float32),
    )(x, deg)


def _l12_body(g_ref, s_ref, dg_ref, b_ref, w1_ref, w2_ref, y_ref):
    dinv = _dinv_block(dg_ref)
    t = dinv * (g_ref[:, :] + s_ref[0] + s_ref[1])
    h = jnp.maximum(
        jnp.dot(t, w1_ref[:, :], preferred_element_type=jnp.float32)
        + b_ref[:, :], 0.0)
    full = dinv * jnp.dot(h, w2_ref[:, :], preferred_element_type=jnp.float32)
    y_ref[0, :, :] = full[:, :HQ]
    y_ref[1, :, :] = full[:, HQ:]


def _l12(g, sagg, deg, b1, W1, W2):
    return pl.pallas_call(
        _l12_body,
        grid=(NBLK,),
        in_specs=[
            pl.BlockSpec((BN, D_IN), lambda n: (n, 0)),
            pl.BlockSpec((2, BN, D_IN), lambda n: (0, n, 0)),
            pl.BlockSpec((2, BN, HQ), lambda n: (0, n, 0)),
            pl.BlockSpec((1, D_HID), lambda n: (0, 0)),
            pl.BlockSpec((D_IN, D_HID), lambda n: (0, 0)),
            pl.BlockSpec((D_HID, D_HID), lambda n: (0, 0)),
        ],
        out_specs=pl.BlockSpec((2, BN, HQ), lambda n: (0, n, 0)),
        out_shape=jax.ShapeDtypeStruct((2, NP, HQ), jnp.float32),
    )(g, sagg, deg, b1.reshape(1, -1), W1, W2)


def _ep_mm_body(y_ref, s_ref, dg_ref, b_ref, w_ref, out_ref):
    dinv = _dinv_block(dg_ref)
    z = jnp.concatenate([y_ref[0] + s_ref[0], y_ref[1] + s_ref[1]], axis=1)
    h = jnp.maximum(z * dinv + b_ref[:, :], 0.0)
    full = dinv * jnp.dot(h, w_ref[:, :], preferred_element_type=jnp.float32)
    if out_ref.shape[0] == 2:
        out_ref[0, :, :] = full[:, :HQ]
        out_ref[1, :, :] = full[:, HQ:]
    else:
        out_ref[:, :] = full


def _ep_mm(y, sagg, deg, b, W):
    d_out = W.shape[1]
    if d_out == 2 * HQ:
        out_spec = pl.BlockSpec((2, BN, HQ), lambda n: (0, n, 0))
        out_shape = jax.ShapeDtypeStruct((2, NP, HQ), jnp.float32)
    else:
        out_spec = pl.BlockSpec((BN, d_out), lambda n: (n, 0))
        out_shape = jax.ShapeDtypeStruct((NP, d_out), jnp.float32)
    return pl.pallas_call(
        _ep_mm_body,
        grid=(NBLK,),
        in_specs=[
            pl.BlockSpec((2, BN, HQ), lambda n: (0, n, 0)),
            pl.BlockSpec((2, BN, HQ), lambda n: (0, n, 0)),
            pl.BlockSpec((2, BN, HQ), lambda n: (0, n, 0)),
            pl.BlockSpec((1, 2 * HQ), lambda n: (0, 0)),
            pl.BlockSpec((2 * HQ, d_out), lambda n: (0, 0)),
        ],
        out_specs=out_spec,
        out_shape=out_shape,
    )(y, sagg, deg, b.reshape(1, -1), W)


def _final_body(y_ref, s_ref, dg_ref, b_ref, out_ref):
    dinv = _dinv_block(dg_ref)
    z = y_ref[:, :] + s_ref[0] + s_ref[1]
    out_ref[:, :] = z * dinv + b_ref[:, :]


def _final(y, sagg, deg, b):
    d_out = y.shape[1]
    return pl.pallas_call(
        _final_body,
        grid=(NBLK,),
        in_specs=[
            pl.BlockSpec((BN, d_out), lambda n: (n, 0)),
            pl.BlockSpec((2, BN, d_out), lambda n: (0, n, 0)),
            pl.BlockSpec((2, BN, HQ), lambda n: (0, n, 0)),
            pl.BlockSpec((1, d_out), lambda n: (0, 0)),
        ],
        out_specs=pl.BlockSpec((BN, d_out), lambda n: (n, 0)),
        out_shape=jax.ShapeDtypeStruct((N, d_out), jnp.float32),
    )(y, sagg, deg, b.reshape(1, -1))


def kernel(x, edge_index, W1, b1, W2, b2, W3, b3):
    ei = edge_index.astype(jnp.int32)
    src, dst = ei[0], ei[1]

    ept12 = E // NS
    ept12_p = 20480
    src_t = src.reshape(NS, ept12)
    dst_t = dst.reshape(NS, ept12)
    srcT12 = _pad_tile_rows(
        jnp.concatenate([src_t, src_t + NP], axis=0), 0, ept12_p)
    dstT12 = _pad_tile_rows(
        jnp.concatenate([dst_t, dst_t], axis=0), N, ept12_p)

    ept3 = E // (NC * NS)
    ept3_p = 10240
    srcT3 = _pad_tile_rows(src.reshape(NC * NS, ept3), 0, ept3_p)
    dstT3 = _pad_tile_rows(dst.reshape(NC * NS, ept3), N, ept3_p)

    deg = _deg_counts(dstT3).reshape(2, NP, HQ)

    g = _gx(x, deg)
    s1 = _edge_aggregate(g, srcT3, dstT3)
    y2 = _l12(g, s1.reshape(2, NP, HQ), deg, b1, W1, W2)
    s2 = _edge_aggregate(y2.reshape(2 * NP, HQ), srcT12, dstT12)
    y3 = _ep_mm(y2, s2.reshape(2, NP, HQ), deg, b2, W3)
    s3 = _edge_aggregate(y3, srcT3, dstT3)
    return _final(y3, s3.reshape(2, NP, HQ), deg, b3)


# speedup vs baseline: 1.3160x; 1.0242x over previous
import functools

import jax
import jax.numpy as jnp
from jax import lax
from jax.experimental import pallas as pl
from jax.experimental.pallas import tpu as pltpu
from jax.experimental.pallas import tpu_sc as plsc

N = 10000
NP = 10240
E = 320000
D_IN, D_HID, D_OUT = 128, 256, 128
HQ = 128

NC, NS = 2, 16
K = 128
KH = 64
SUP = 8
IRS = 2
NRING = 4
RPT = NP // NS

BN = 1000
NBLK = N // BN


def _sc_mesh():
    return plsc.VectorSubcoreMesh(
        core_axis_name="c", subcore_axis_name="s", num_cores=NC, num_subcores=NS
    )


def _pad_tile_rows(a, fill, ept_p):
    t, ept = a.shape
    pad = ept_p - ept
    a = jnp.concatenate(
        [a, jnp.full((t, pad), fill, jnp.int32)], axis=1)
    return a.reshape(t, ept_p // (SUP * K), SUP, K)


def _edge_body(y_hbm, srcT, dstT, zrows_hbm, out_hbm, sring, dring, buf, acc,
               issem, idsem, gsem, *, nsup):
    c = lax.axis_index("c")
    s = lax.axis_index("s")
    w = c * NS + s
    nhp = 2 * SUP

    def half(ring, su, p):
        return ring.at[su].at[p // 2].at[pl.ds((p % 2) * KH, KH)]

    pltpu.sync_copy(zrows_hbm, acc.at[pl.ds(s * RPT, RPT)])

    for r in range(IRS):
        pltpu.async_copy(srcT.at[w].at[r], sring.at[r], issem.at[r])
        pltpu.async_copy(dstT.at[w].at[r], dring.at[r], idsem.at[r])
    plsc.subcore_barrier()

    pltpu.make_async_copy(srcT.at[w].at[0], sring.at[0], issem.at[0]).wait()
    for b in range(NRING):
        pltpu.async_copy(y_hbm.at[half(sring, 0, b)], buf.at[b], gsem.at[b])

    def super_body(u, _):
        su = lax.rem(u, IRS)
        pltpu.make_async_copy(dstT.at[w].at[u], dring.at[su], idsem.at[su]).wait()
        for p in range(nhp):
            m = p % NRING
            pltpu.make_async_copy(y_hbm.at[half(sring, su, p)], buf.at[m],
                                  gsem.at[m]).wait()
            pltpu.sync_copy(buf.at[m], acc.at[half(dring, su, p)], add=True)

            if p == nhp - NRING:
                @pl.when(u + 1 < nsup)
                def _():
                    pltpu.make_async_copy(srcT.at[w].at[u + 1],
                                          sring.at[lax.rem(u + 1, IRS)],
                                          issem.at[lax.rem(u + 1, IRS)]).wait()

            pn = p + NRING
            if pn < nhp:
                pltpu.async_copy(y_hbm.at[half(sring, su, pn)], buf.at[m],
                                 gsem.at[m])
            else:
                @pl.when(u + 1 < nsup)
                def _():
                    pltpu.async_copy(
                        y_hbm.at[half(sring, lax.rem(u + 1, IRS), pn - nhp)],
                        buf.at[m], gsem.at[m])

        @pl.when(u + IRS < nsup)
        def _():
            pltpu.async_copy(srcT.at[w].at[u + IRS], sring.at[su],
                             issem.at[su])
            pltpu.async_copy(dstT.at[w].at[u + IRS], dring.at[su],
                             idsem.at[su])
        return 0

    lax.fori_loop(0, nsup, super_body, 0)
    plsc.subcore_barrier()

    pltpu.sync_copy(acc.at[pl.ds(s * RPT, RPT)],
                    out_hbm.at[pl.ds(c * NP + s * RPT, RPT)])


def _edge_aggregate(y_table, srcT, dstT):
    nsup = srcT.shape[1]
    kern = pl.kernel(
        functools.partial(_edge_body, nsup=nsup),
        out_type=jax.ShapeDtypeStruct((2 * NP, HQ), jnp.float32),
        mesh=_sc_mesh(),
        scratch_types=[
            pltpu.VMEM((IRS, SUP, K), jnp.int32),
            pltpu.VMEM((IRS, SUP, K), jnp.int32),
            pltpu.VMEM((NRING, KH, HQ), jnp.float32),
            pltpu.VMEM_SHARED((NP, HQ), jnp.float32),
            pltpu.SemaphoreType.DMA((IRS,)),
            pltpu.SemaphoreType.DMA((IRS,)),
            pltpu.SemaphoreType.DMA((NRING,)),
        ],
    )
    zrows = jnp.zeros((RPT, HQ), jnp.float32)
    return kern(y_table, srcT, dstT, zrows)


def _deg_body(dstT, ones_hbm, zrows_hbm, deg_hbm, dring, ones_v, acc, idsem,
              *, nsup):
    c = lax.axis_index("c")
    s = lax.axis_index("s")
    w = c * NS + s

    pltpu.sync_copy(zrows_hbm, acc.at[pl.ds(s * RPT, RPT)])
    pltpu.sync_copy(ones_hbm, ones_v)
    for r in range(IRS):
        pltpu.async_copy(dstT.at[w].at[r], dring.at[r], idsem.at[r])
    plsc.subcore_barrier()

    def super_body(u, _):
        su = lax.rem(u, IRS)
        pltpu.make_async_copy(dstT.at[w].at[u], dring.at[su], idsem.at[su]).wait()
        for j in range(SUP):
            pltpu.sync_copy(ones_v, acc.at[dring.at[su].at[j]], add=True)

        @pl.when(u + IRS < nsup)
        def _():
            pltpu.async_copy(dstT.at[w].at[u + IRS], dring.at[su], idsem.at[su])
        return 0

    lax.fori_loop(0, nsup, super_body, 0)
    plsc.subcore_barrier()

    pltpu.sync_copy(acc.at[pl.ds(s * RPT, RPT)],
                    deg_hbm.at[pl.ds(c * NP + s * RPT, RPT)])


def _deg_counts(dstT):
    nsup = dstT.shape[1]
    kern = pl.kernel(
        functools.partial(_deg_body, nsup=nsup),
        out_type=jax.ShapeDtypeStruct((2 * NP, HQ), jnp.float32),
        mesh=_sc_mesh(),
        scratch_types=[
            pltpu.VMEM((IRS, SUP, K), jnp.int32),
            pltpu.VMEM((K, HQ), jnp.float32),
            pltpu.VMEM_SHARED((NP, HQ), jnp.float32),
            pltpu.SemaphoreType.DMA((IRS,)),
        ],
    )
    ones = jnp.ones((K, HQ), jnp.float32)
    zrows = jnp.zeros((RPT, HQ), jnp.float32)
    return kern(dstT, ones, zrows)


def _dinv_block(dg_ref):
    deg = dg_ref[0, :, 0:1] + dg_ref[1, :, 0:1] + 1.0
    return lax.rsqrt(deg)


def _gx_body(x_ref, dg_ref, g_ref):
    g_ref[:, :] = _dinv_block(dg_ref) * x_ref[:, :]


def _gx(x, deg):
    return pl.pallas_call(
        _gx_body,
        grid=(NBLK,),
        in_specs=[
            pl.BlockSpec((BN, D_IN), lambda n: (n, 0)),
            pl.BlockSpec((2, BN, HQ), lambda n: (0, n, 0)),
        ],
        out_specs=pl.BlockSpec((BN, D_IN), lambda n: (n, 0)),
        out_shape=jax.ShapeDtypeStruct((NP, D_IN), jnp.float32),
    )(x, deg)


def _l12_body(g_ref, s_ref, dg_ref, b_ref, w1_ref, w2_ref, y_ref):
    dinv = _dinv_block(dg_ref)
    t = dinv * (g_ref[:, :] + s_ref[0] + s_ref[1])
    h = jnp.maximum(
        jnp.dot(t, w1_ref[:, :], preferred_element_type=jnp.float32)
        + b_ref[:, :], 0.0)
    full = dinv * jnp.dot(h, w2_ref[:, :], preferred_element_type=jnp.float32)
    y_ref[0, :, :] = full[:, :HQ]
    y_ref[1, :, :] = full[:, HQ:]


def _l12(g, sagg, deg, b1, W1, W2):
    return pl.pallas_call(
        _l12_body,
        grid=(NBLK,),
        in_specs=[
            pl.BlockSpec((BN, D_IN), lambda n: (n, 0)),
            pl.BlockSpec((2, BN, D_IN), lambda n: (0, n, 0)),
            pl.BlockSpec((2, BN, HQ), lambda n: (0, n, 0)),
            pl.BlockSpec((1, D_HID), lambda n: (0, 0)),
            pl.BlockSpec((D_IN, D_HID), lambda n: (0, 0)),
            pl.BlockSpec((D_HID, D_HID), lambda n: (0, 0)),
        ],
        out_specs=pl.BlockSpec((2, BN, HQ), lambda n: (0, n, 0)),
        out_shape=jax.ShapeDtypeStruct((2, NP, HQ), jnp.float32),
    )(g, sagg, deg, b1.reshape(1, -1), W1, W2)


def _ep_mm_body(y_ref, s_ref, dg_ref, b_ref, w_ref, out_ref):
    dinv = _dinv_block(dg_ref)
    z = jnp.concatenate([y_ref[0] + s_ref[0], y_ref[1] + s_ref[1]], axis=1)
    h = jnp.maximum(z * dinv + b_ref[:, :], 0.0)
    full = dinv * jnp.dot(h, w_ref[:, :], preferred_element_type=jnp.float32)
    if out_ref.shape[0] == 2:
        out_ref[0, :, :] = full[:, :HQ]
        out_ref[1, :, :] = full[:, HQ:]
    else:
        out_ref[:, :] = full


def _ep_mm(y, sagg, deg, b, W):
    d_out = W.shape[1]
    if d_out == 2 * HQ:
        out_spec = pl.BlockSpec((2, BN, HQ), lambda n: (0, n, 0))
        out_shape = jax.ShapeDtypeStruct((2, NP, HQ), jnp.float32)
    else:
        out_spec = pl.BlockSpec((BN, d_out), lambda n: (n, 0))
        out_shape = jax.ShapeDtypeStruct((NP, d_out), jnp.float32)
    return pl.pallas_call(
        _ep_mm_body,
        grid=(NBLK,),
        in_specs=[
            pl.BlockSpec((2, BN, HQ), lambda n: (0, n, 0)),
            pl.BlockSpec((2, BN, HQ), lambda n: (0, n, 0)),
            pl.BlockSpec((2, BN, HQ), lambda n: (0, n, 0)),
            pl.BlockSpec((1, 2 * HQ), lambda n: (0, 0)),
            pl.BlockSpec((2 * HQ, d_out), lambda n: (0, 0)),
        ],
        out_specs=out_spec,
        out_shape=out_shape,
    )(y, sagg, deg, b.reshape(1, -1), W)


def _final_body(y_ref, s_ref, dg_ref, b_ref, out_ref):
    dinv = _dinv_block(dg_ref)
    z = y_ref[:, :] + s_ref[0] + s_ref[1]
    out_ref[:, :] = z * dinv + b_ref[:, :]


def _final(y, sagg, deg, b):
    d_out = y.shape[1]
    return pl.pallas_call(
        _final_body,
        grid=(NBLK,),
        in_specs=[
            pl.BlockSpec((BN, d_out), lambda n: (n, 0)),
            pl.BlockSpec((2, BN, d_out), lambda n: (0, n, 0)),
            pl.BlockSpec((2, BN, HQ), lambda n: (0, n, 0)),
            pl.BlockSpec((1, d_out), lambda n: (0, 0)),
        ],
        out_specs=pl.BlockSpec((BN, d_out), lambda n: (n, 0)),
        out_shape=jax.ShapeDtypeStruct((N, d_out), jnp.float32),
    )(y, sagg, deg, b.reshape(1, -1))


def kernel(x, edge_index, W1, b1, W2, b2, W3, b3):
    ei = edge_index.astype(jnp.int32)
    src, dst = ei[0], ei[1]

    ept12 = E // NS
    ept12_p = 20480
    src_t = src.reshape(NS, ept12)
    dst_t = dst.reshape(NS, ept12)
    srcT12 = _pad_tile_rows(
        jnp.concatenate([src_t, src_t + NP], axis=0), 0, ept12_p)
    dstT12 = _pad_tile_rows(
        jnp.concatenate([dst_t, dst_t], axis=0), N, ept12_p)

    ept3 = E // (NC * NS)
    ept3_p = 10240
    srcT3 = _pad_tile_rows(src.reshape(NC * NS, ept3), 0, ept3_p)
    dstT3 = _pad_tile_rows(dst.reshape(NC * NS, ept3), N, ept3_p)

    deg = _deg_counts(dstT3).reshape(2, NP, HQ)

    g = _gx(x, deg)
    s1 = _edge_aggregate(g, srcT3, dstT3)
    y2 = _l12(g, s1.reshape(2, NP, HQ), deg, b1, W1, W2)
    s2 = _edge_aggregate(y2.reshape(2 * NP, HQ), srcT12, dstT12)
    y3 = _ep_mm(y2, s2.reshape(2, NP, HQ), deg, b2, W3)
    s3 = _edge_aggregate(y3, srcT3, dstT3)
    return _final(y3, s3.reshape(2, NP, HQ), deg, b3)


# speedup vs baseline: 1.3179x; 1.0014x over previous
import functools

import jax
import jax.numpy as jnp
from jax import lax
from jax.experimental import pallas as pl
from jax.experimental.pallas import tpu as pltpu
from jax.experimental.pallas import tpu_sc as plsc

N = 10000
NP = 10240
E = 320000
D_IN, D_HID, D_OUT = 128, 256, 128
HQ = 128

NC, NS = 2, 16
K = 128
KH = 32
SUP = 8
IRS = 2
NRING = 8
RPT = NP // NS

BN = 1000
NBLK = N // BN


def _sc_mesh():
    return plsc.VectorSubcoreMesh(
        core_axis_name="c", subcore_axis_name="s", num_cores=NC, num_subcores=NS
    )


def _pad_tile_rows(a, fill, ept_p):
    t, ept = a.shape
    pad = ept_p - ept
    a = jnp.concatenate(
        [a, jnp.full((t, pad), fill, jnp.int32)], axis=1)
    return a.reshape(t, ept_p // (SUP * K), SUP, K)


def _edge_body(y_hbm, srcT, dstT, zrows_hbm, out_hbm, sring, dring, buf, acc,
               issem, idsem, gsem, *, nsup):
    c = lax.axis_index("c")
    s = lax.axis_index("s")
    w = c * NS + s
    ph = K // KH
    nhp = ph * SUP

    def half(ring, su, p):
        return ring.at[su].at[p // ph].at[pl.ds((p % ph) * KH, KH)]

    pltpu.sync_copy(zrows_hbm, acc.at[pl.ds(s * RPT, RPT)])

    for r in range(IRS):
        pltpu.async_copy(srcT.at[w].at[r], sring.at[r], issem.at[r])
        pltpu.async_copy(dstT.at[w].at[r], dring.at[r], idsem.at[r])
    plsc.subcore_barrier()

    pltpu.make_async_copy(srcT.at[w].at[0], sring.at[0], issem.at[0]).wait()
    for b in range(NRING):
        pltpu.async_copy(y_hbm.at[half(sring, 0, b)], buf.at[b], gsem.at[b])

    def super_body(u, _):
        su = lax.rem(u, IRS)
        pltpu.make_async_copy(dstT.at[w].at[u], dring.at[su], idsem.at[su]).wait()
        for p in range(nhp):
            m = p % NRING
            pltpu.make_async_copy(y_hbm.at[half(sring, su, p)], buf.at[m],
                                  gsem.at[m]).wait()
            pltpu.sync_copy(buf.at[m], acc.at[half(dring, su, p)], add=True)

            if p == nhp - NRING:
                @pl.when(u + 1 < nsup)
                def _():
                    pltpu.make_async_copy(srcT.at[w].at[u + 1],
                                          sring.at[lax.rem(u + 1, IRS)],
                                          issem.at[lax.rem(u + 1, IRS)]).wait()

            pn = p + NRING
            if pn < nhp:
                pltpu.async_copy(y_hbm.at[half(sring, su, pn)], buf.at[m],
                                 gsem.at[m])
            else:
                @pl.when(u + 1 < nsup)
                def _():
                    pltpu.async_copy(
                        y_hbm.at[half(sring, lax.rem(u + 1, IRS), pn - nhp)],
                        buf.at[m], gsem.at[m])

        @pl.when(u + IRS < nsup)
        def _():
            pltpu.async_copy(srcT.at[w].at[u + IRS], sring.at[su],
                             issem.at[su])
            pltpu.async_copy(dstT.at[w].at[u + IRS], dring.at[su],
                             idsem.at[su])
        return 0

    lax.fori_loop(0, nsup, super_body, 0)
    plsc.subcore_barrier()

    pltpu.sync_copy(acc.at[pl.ds(s * RPT, RPT)],
                    out_hbm.at[pl.ds(c * NP + s * RPT, RPT)])


def _edge_aggregate(y_table, srcT, dstT):
    nsup = srcT.shape[1]
    kern = pl.kernel(
        functools.partial(_edge_body, nsup=nsup),
        out_type=jax.ShapeDtypeStruct((2 * NP, HQ), jnp.float32),
        mesh=_sc_mesh(),
        scratch_types=[
            pltpu.VMEM((IRS, SUP, K), jnp.int32),
            pltpu.VMEM((IRS, SUP, K), jnp.int32),
            pltpu.VMEM((NRING, KH, HQ), jnp.float32),
            pltpu.VMEM_SHARED((NP, HQ), jnp.float32),
            pltpu.SemaphoreType.DMA((IRS,)),
            pltpu.SemaphoreType.DMA((IRS,)),
            pltpu.SemaphoreType.DMA((NRING,)),
        ],
    )
    zrows = jnp.zeros((RPT, HQ), jnp.float32)
    return kern(y_table, srcT, dstT, zrows)


def _deg_body(dstT, ones_hbm, zrows_hbm, deg_hbm, dring, ones_v, acc, idsem,
              *, nsup):
    c = lax.axis_index("c")
    s = lax.axis_index("s")
    w = c * NS + s

    pltpu.sync_copy(zrows_hbm, acc.at[pl.ds(s * RPT, RPT)])
    pltpu.sync_copy(ones_hbm, ones_v)
    for r in range(IRS):
        pltpu.async_copy(dstT.at[w].at[r], dring.at[r], idsem.at[r])
    plsc.subcore_barrier()

    def super_body(u, _):
        su = lax.rem(u, IRS)
        pltpu.make_async_copy(dstT.at[w].at[u], dring.at[su], idsem.at[su]).wait()
        for j in range(SUP):
            pltpu.sync_copy(ones_v, acc.at[dring.at[su].at[j]], add=True)

        @pl.when(u + IRS < nsup)
        def _():
            pltpu.async_copy(dstT.at[w].at[u + IRS], dring.at[su], idsem.at[su])
        return 0

    lax.fori_loop(0, nsup, super_body, 0)
    plsc.subcore_barrier()

    pltpu.sync_copy(acc.at[pl.ds(s * RPT, RPT)],
                    deg_hbm.at[pl.ds(c * NP + s * RPT, RPT)])


def _deg_counts(dstT):
    nsup = dstT.shape[1]
    kern = pl.kernel(
        functools.partial(_deg_body, nsup=nsup),
        out_type=jax.ShapeDtypeStruct((2 * NP, HQ), jnp.float32),
        mesh=_sc_mesh(),
        scratch_types=[
            pltpu.VMEM((IRS, SUP, K), jnp.int32),
            pltpu.VMEM((K, HQ), jnp.float32),
            pltpu.VMEM_SHARED((NP, HQ), jnp.float32),
            pltpu.SemaphoreType.DMA((IRS,)),
        ],
    )
    ones = jnp.ones((K, HQ), jnp.float32)
    zrows = jnp.zeros((RPT, HQ), jnp.float32)
    return kern(dstT, ones, zrows)


def _dinv_block(dg_ref):
    deg = dg_ref[0, :, 0:1] + dg_ref[1, :, 0:1] + 1.0
    return lax.rsqrt(deg)


def _gx_body(x_ref, dg_ref, g_ref):
    g_ref[:, :] = _dinv_block(dg_ref) * x_ref[:, :]


def _gx(x, deg):
    return pl.pallas_call(
        _gx_body,
        grid=(NBLK,),
        in_specs=[
            pl.BlockSpec((BN, D_IN), lambda n: (n, 0)),
            pl.BlockSpec((2, BN, HQ), lambda n: (0, n, 0)),
        ],
        out_specs=pl.BlockSpec((BN, D_IN), lambda n: (n, 0)),
        out_shape=jax.ShapeDtypeStruct((NP, D_IN), jnp.float32),
    )(x, deg)


def _l12_body(g_ref, s_ref, dg_ref, b_ref, w1_ref, w2_ref, y_ref):
    dinv = _dinv_block(dg_ref)
    t = dinv * (g_ref[:, :] + s_ref[0] + s_ref[1])
    h = jnp.maximum(
        jnp.dot(t, w1_ref[:, :], preferred_element_type=jnp.float32)
        + b_ref[:, :], 0.0)
    full = dinv * jnp.dot(h, w2_ref[:, :], preferred_element_type=jnp.float32)
    y_ref[0, :, :] = full[:, :HQ]
    y_ref[1, :, :] = full[:, HQ:]


def _l12(g, sagg, deg, b1, W1, W2):
    return pl.pallas_call(
        _l12_body,
        grid=(NBLK,),
        in_specs=[
            pl.BlockSpec((BN, D_IN), lambda n: (n, 0)),
            pl.BlockSpec((2, BN, D_IN), lambda n: (0, n, 0)),
            pl.BlockSpec((2, BN, HQ), lambda n: (0, n, 0)),
            pl.BlockSpec((1, D_HID), lambda n: (0, 0)),
            pl.BlockSpec((D_IN, D_HID), lambda n: (0, 0)),
            pl.BlockSpec((D_HID, D_HID), lambda n: (0, 0)),
        ],
        out_specs=pl.BlockSpec((2, BN, HQ), lambda n: (0, n, 0)),
        out_shape=jax.ShapeDtypeStruct((2, NP, HQ), jnp.float32),
    )(g, sagg, deg, b1.reshape(1, -1), W1, W2)


def _ep_mm_body(y_ref, s_ref, dg_ref, b_ref, w_ref, out_ref):
    dinv = _dinv_block(dg_ref)
    z = jnp.concatenate([y_ref[0] + s_ref[0], y_ref[1] + s_ref[1]], axis=1)
    h = jnp.maximum(z * dinv + b_ref[:, :], 0.0)
    full = dinv * jnp.dot(h, w_ref[:, :], preferred_element_type=jnp.float32)
    if out_ref.shape[0] == 2:
        out_ref[0, :, :] = full[:, :HQ]
        out_ref[1, :, :] = full[:, HQ:]
    else:
        out_ref[:, :] = full


def _ep_mm(y, sagg, deg, b, W):
    d_out = W.shape[1]
    if d_out == 2 * HQ:
        out_spec = pl.BlockSpec((2, BN, HQ), lambda n: (0, n, 0))
        out_shape = jax.ShapeDtypeStruct((2, NP, HQ), jnp.float32)
    else:
        out_spec = pl.BlockSpec((BN, d_out), lambda n: (n, 0))
        out_shape = jax.ShapeDtypeStruct((NP, d_out), jnp.float32)
    return pl.pallas_call(
        _ep_mm_body,
        grid=(NBLK,),
        in_specs=[
            pl.BlockSpec((2, BN, HQ), lambda n: (0, n, 0)),
            pl.BlockSpec((2, BN, HQ), lambda n: (0, n, 0)),
            pl.BlockSpec((2, BN, HQ), lambda n: (0, n, 0)),
            pl.BlockSpec((1, 2 * HQ), lambda n: (0, 0)),
            pl.BlockSpec((2 * HQ, d_out), lambda n: (0, 0)),
        ],
        out_specs=out_spec,
        out_shape=out_shape,
    )(y, sagg, deg, b.reshape(1, -1), W)


def _final_body(y_ref, s_ref, dg_ref, b_ref, out_ref):
    dinv = _dinv_block(dg_ref)
    z = y_ref[:, :] + s_ref[0] + s_ref[1]
    out_ref[:, :] = z * dinv + b_ref[:, :]


def _final(y, sagg, deg, b):
    d_out = y.shape[1]
    return pl.pallas_call(
        _final_body,
        grid=(NBLK,),
        in_specs=[
            pl.BlockSpec((BN, d_out), lambda n: (n, 0)),
            pl.BlockSpec((2, BN, d_out), lambda n: (0, n, 0)),
            pl.BlockSpec((2, BN, HQ), lambda n: (0, n, 0)),
            pl.BlockSpec((1, d_out), lambda n: (0, 0)),
        ],
        out_specs=pl.BlockSpec((BN, d_out), lambda n: (n, 0)),
        out_shape=jax.ShapeDtypeStruct((N, d_out), jnp.float32),
    )(y, sagg, deg, b.reshape(1, -1))


def kernel(x, edge_index, W1, b1, W2, b2, W3, b3):
    ei = edge_index.astype(jnp.int32)
    src, dst = ei[0], ei[1]

    ept12 = E // NS
    ept12_p = 20480
    src_t = src.reshape(NS, ept12)
    dst_t = dst.reshape(NS, ept12)
    srcT12 = _pad_tile_rows(
        jnp.concatenate([src_t, src_t + NP], axis=0), 0, ept12_p)
    dstT12 = _pad_tile_rows(
        jnp.concatenate([dst_t, dst_t], axis=0), N, ept12_p)

    ept3 = E // (NC * NS)
    ept3_p = 10240
    srcT3 = _pad_tile_rows(src.reshape(NC * NS, ept3), 0, ept3_p)
    dstT3 = _pad_tile_rows(dst.reshape(NC * NS, ept3), N, ept3_p)

    deg = _deg_counts(dstT3).reshape(2, NP, HQ)

    g = _gx(x, deg)
    s1 = _edge_aggregate(g, srcT3, dstT3)
    y2 = _l12(g, s1.reshape(2, NP, HQ), deg, b1, W1, W2)
    s2 = _edge_aggregate(y2.reshape(2 * NP, HQ), srcT12, dstT12)
    y3 = _ep_mm(y2, s2.reshape(2, NP, HQ), deg, b2, W3)
    s3 = _edge_aggregate(y3, srcT3, dstT3)
    return _final(y3, s3.reshape(2, NP, HQ), deg, b3)
